# Initial kernel scaffold; baseline (speedup 1.0000x reference)
import jax
import jax.numpy as jnp
from jax.experimental import pallas as pl


def kernel(inputs, input_masks, edge_index, Wq, Wk, wv, Wfc, bfc):
    raise NotImplementedError("write your pallas kernel here")


# speedup vs baseline: 36.7587x; 36.7587x over previous
import jax
import jax.numpy as jnp
import numpy as np
from jax import lax
from jax.experimental import pallas as pl
from jax.experimental.pallas import tpu as pltpu

_B = 8


def _unpack(p8, s):
    return jnp.concatenate([p8[:, a * s:(a + 1) * s] for a in range(4)], axis=0)


def _pack(v, s):
    return jnp.concatenate([v[a * 8:(a + 1) * 8, :] for a in range(4)], axis=1)


def _gat_body(in_hbm, mf_ref, seg_ref, ei_ref, conf_ref, pconf_ref, wqt_ref,
              wqrep_ref, k4_ref, rep_ref, swvp_ref, wfcpp_ref, bfcp_ref,
              colp_ref, out_hbm, table, vls_ref, sem):
    n_nodes = table.shape[0]
    s = colp_ref.shape[0]
    n_edges = ei_ref.shape[1]
    n_blocks = n_edges // _B

    cp_in = pltpu.make_async_copy(in_hbm, table, sem)
    cp_in.start()
    vls_ref[...] = jnp.dot(mf_ref[...], seg_ref[...],
                           preferred_element_type=jnp.float32)
    rows = n_nodes // 128
    nidx = (lax.broadcasted_iota(jnp.int32, (rows, 128), 0) * 128
            + lax.broadcasted_iota(jnp.int32, (rows, 128), 1))
    cp_in.wait()

    def node_vl(node):
        return jnp.sum(jnp.where(nidx == node, vls_ref[...], 0.0))

    def attend(fqp, fk8, vp, vl):
        fq_b = jnp.dot(fqp, rep_ref[...], preferred_element_type=jnp.float32)
        ones_col = jnp.ones((s, 1), dtype=jnp.float32)
        fk_b = jnp.concatenate(
            [jnp.dot(ones_col, fk8[t:t + 1, :],
                     preferred_element_type=jnp.float32) for t in range(8)],
            axis=1)
        t = jnp.tanh(fq_b + fk_b)
        scores = jnp.dot(t, swvp_ref[...], preferred_element_type=jnp.float32)
        scores = jnp.where(colp_ref[...] < vl, scores, -1e6)
        m = jnp.max(scores, axis=-1, keepdims=True)
        e = jnp.exp(scores - m)
        unnorm = jnp.dot(e, vp, preferred_element_type=jnp.float32)
        return unnorm * (1.0 / jnp.sum(e, axis=-1, keepdims=True))

    def edge_outputs(ex8, ey8, exp_, eyp_, vlx, vly):
        fqxp = jnp.dot(exp_, wqt_ref[...], preferred_element_type=jnp.float32)
        fqyp = jnp.dot(eyp_, wqt_ref[...], preferred_element_type=jnp.float32)
        fk8x = jnp.dot(ex8, k4_ref[...], preferred_element_type=jnp.float32)
        fk8y = jnp.dot(ey8, k4_ref[...], preferred_element_type=jnp.float32)

        y2x = attend(fqxp, fk8y, eyp_, vlx)
        x2y = attend(fqyp, fk8x, exp_, vly)

        xin = jnp.concatenate([exp_, y2x], axis=0)
        yin = jnp.concatenate([eyp_, x2y], axis=0)
        x_out = jnp.dot(wfcpp_ref[...], xin,
                        preferred_element_type=jnp.float32) + bfcp_ref[...]
        y_out = jnp.dot(wfcpp_ref[...], yin,
                        preferred_element_type=jnp.float32) + bfcp_ref[...]
        return x_out, y_out

    def seq_step(e, carry):
        x = ei_ref[0, e]
        y = ei_ref[1, e]
        x_out, y_out = edge_outputs(
            table[x], table[y], _unpack(table[x], s), _unpack(table[y], s),
            node_vl(x), node_vl(y))
        table[x] = _pack(x_out, s)
        table[y] = _pack(y_out, s)
        return carry

    def batched_window(e0, w):
        xs = [ei_ref[0, e0 + i] for i in range(w)]
        ys = [ei_ref[1, e0 + i] for i in range(w)]
        nodes = []
        for i in range(w):
            nodes.append(xs[i])
            nodes.append(ys[i])
        tiles = [table[nd] for nd in nodes]
        vls = [node_vl(nd) for nd in nodes]
        unps = [_unpack(t, s) for t in tiles]
        stk = jnp.concatenate(tiles, axis=0)
        ud = jnp.concatenate(unps, axis=0)
        fqb = jnp.dot(ud, wqrep_ref[...],
                      preferred_element_type=jnp.float32)
        fk8 = jnp.dot(stk, k4_ref[...],
                      preferred_element_type=jnp.float32)
        ones_col = jnp.ones((s, 1), dtype=jnp.float32)
        fkb = jnp.concatenate([
            jnp.concatenate(
                [jnp.dot(ones_col, fk8[8 * (d ^ 1) + u:8 * (d ^ 1) + u + 1, :],
                         preferred_element_type=jnp.float32)
                 for u in range(8)], axis=1)
            for d in range(2 * w)], axis=0)
        t = jnp.tanh(fqb + fkb)
        sc = jnp.dot(t, swvp_ref[...],
                     preferred_element_type=jnp.float32)
        atts = []
        for d in range(2 * w):
            scores = jnp.where(colp_ref[...] < vls[d],
                               sc[d * s:(d + 1) * s, :], -1e6)
            m = jnp.max(scores, axis=-1, keepdims=True)
            e = jnp.exp(scores - m)
            unnorm = jnp.dot(e, unps[d ^ 1],
                             preferred_element_type=jnp.float32)
            atts.append(unnorm * (1.0 / jnp.sum(e, axis=-1, keepdims=True)))
        for i in range(w):
            xin = jnp.concatenate([unps[2 * i], atts[2 * i]], axis=0)
            yin = jnp.concatenate([unps[2 * i + 1], atts[2 * i + 1]], axis=0)
            x_out = jnp.dot(wfcpp_ref[...], xin,
                            preferred_element_type=jnp.float32) + bfcp_ref[...]
            y_out = jnp.dot(wfcpp_ref[...], yin,
                            preferred_element_type=jnp.float32) + bfcp_ref[...]
            table[xs[i]] = _pack(x_out, s)
            table[ys[i]] = _pack(y_out, s)

    def pair_step(j, carry):
        e0 = j * 2 * _B

        @pl.when(pconf_ref[0, j] == 0)
        def _fused():
            batched_window(e0, 2 * _B)

        @pl.when(pconf_ref[0, j] != 0)
        def _split():
            for h in range(2):
                b = 2 * j + h
                eh = e0 + h * _B

                @pl.when(conf_ref[0, b] == 0)
                def _batched(eh=eh):
                    batched_window(eh, _B)

                @pl.when(conf_ref[0, b] != 0)
                def _sequential(eh=eh):
                    lax.fori_loop(eh, eh + _B, seq_step, 0)

        return carry

    lax.fori_loop(0, n_blocks // 2, pair_step, 0)

    cp_out = pltpu.make_async_copy(table, out_hbm, sem)
    cp_out.start()
    cp_out.wait()


def kernel(inputs, input_masks, edge_index, Wq, Wk, wv, Wfc, bfc):
    n, s, _ = inputs.shape
    n_edges = edge_index.shape[1]
    n_blocks = n_edges // _B
    inputs3 = inputs.reshape(n, 8, (s * s) // 8)

    perm = np.array([4 * (i % 8) + i // 8 for i in range(s)])
    eye = np.eye(s, dtype=np.float32)
    k4 = jnp.kron(jnp.eye(4, dtype=Wk.dtype), Wk.T)
    rep = jnp.asarray(np.tile(eye, (1, s)))
    swvp = jnp.kron(jnp.eye(s, dtype=wv.dtype), wv)[:, perm]
    colp = jnp.asarray(np.broadcast_to(perm[None, :], (s, s)).astype(np.float32))
    colperm = np.concatenate([perm, s + perm])
    wfcpp = Wfc[perm][:, colperm]
    bfcp = bfc[perm].reshape(s, 1)

    mf = input_masks.astype(jnp.float32).reshape(n // 128, 128 * s)
    seg = jnp.asarray(np.kron(np.eye(128, dtype=np.float32),
                              np.ones((s, 1), dtype=np.float32)))

    def conflicts(w):
        ids = edge_index.T.reshape(n_edges // w, 2 * w)
        pair_eq = ids[:, :, None] == ids[:, None, :]
        u = np.arange(2 * w)
        cross = (u[:, None] < u[None, :]) & (u[:, None] // 2 != u[None, :] // 2)
        c = jnp.any(pair_eq & jnp.asarray(cross)[None], axis=(1, 2))
        return c.astype(jnp.int32).reshape(1, n_edges // w)

    conf = conflicts(_B)
    pconf = conflicts(2 * _B)

    out = pl.pallas_call(
        _gat_body,
        out_shape=jax.ShapeDtypeStruct(inputs3.shape, inputs3.dtype),
        in_specs=[
            pl.BlockSpec(memory_space=pl.ANY),
            pl.BlockSpec(memory_space=pltpu.VMEM),
            pl.BlockSpec(memory_space=pltpu.VMEM),
            pl.BlockSpec(memory_space=pltpu.SMEM),
            pl.BlockSpec(memory_space=pltpu.SMEM),
            pl.BlockSpec(memory_space=pltpu.SMEM),
            pl.BlockSpec(memory_space=pltpu.VMEM),
            pl.BlockSpec(memory_space=pltpu.VMEM),
            pl.BlockSpec(memory_space=pltpu.VMEM),
            pl.BlockSpec(memory_space=pltpu.VMEM),
            pl.BlockSpec(memory_space=pltpu.VMEM),
            pl.BlockSpec(memory_space=pltpu.VMEM),
            pl.BlockSpec(memory_space=pltpu.VMEM),
            pl.BlockSpec(memory_space=pltpu.VMEM),
        ],
        out_specs=pl.BlockSpec(memory_space=pl.ANY),
        scratch_shapes=[
            pltpu.VMEM(inputs3.shape, inputs3.dtype),
            pltpu.VMEM((n // 128, 128), jnp.float32),
            pltpu.SemaphoreType.DMA,
        ],
    )(inputs3, mf, seg, edge_index, conf, pconf, Wq.T, jnp.tile(Wq.T, (1, s)),
      k4, rep, swvp, wfcpp, bfcp, colp)
    return out.reshape(n, s, s)
